# baseline (speedup 1.0000x reference)
import jax
import jax.numpy as jnp
from jax import lax
from jax.experimental import pallas as pl
from jax.experimental.pallas import tpu as pltpu

_T = 128
_CIN = 9
_C1 = 18
_C2 = 36
_HID = 128
_G = 8
_NS = _T // _G
_T4 = _T // 4
_NCLS = 6


def _har_kernel(x_ref, w1ma_ref, w1pa_ref, w1mb_ref, w1nb_ref, b1_ref,
                w2ma_ref, w2pa_ref, w2mb_ref, w2nb_ref, b2_ref,
                wf1_ref, bf1_ref, wf2_ref, bf2_ref, out_ref):
    bb = x_ref.shape[0]
    f32 = jnp.float32

    xw = x_ref[...].astype(f32)
    xs = [xw[:, 72 * s:72 * (s + 1)] for s in range(_NS)]

    p1 = []
    for s in range(_NS):
        acc_a = jnp.dot(xs[s], w1ma_ref[...], preferred_element_type=f32)
        if s > 0:
            acc_a = acc_a + jnp.dot(xs[s - 1][:, 63:72], w1pa_ref[...],
                                    preferred_element_type=f32)
        acc_b = jnp.dot(xs[s], w1mb_ref[...], preferred_element_type=f32)
        if s < _NS - 1:
            acc_b = acc_b + jnp.dot(xs[s + 1][:, 0:9], w1nb_ref[...],
                                    preferred_element_type=f32)
        a = jnp.maximum(acc_a + b1_ref[...], 0.0)
        bd = jnp.maximum(acc_b + b1_ref[...], 0.0)
        p1.append(jnp.maximum(a, bd))

    p2 = []
    for s in range(_NS):
        acc_a = jnp.dot(p1[s], w2ma_ref[...], preferred_element_type=f32)
        if s > 0:
            acc_a = acc_a + jnp.dot(p1[s - 1][:, 54:72], w2pa_ref[...],
                                    preferred_element_type=f32)
        acc_b = jnp.dot(p1[s], w2mb_ref[...], preferred_element_type=f32)
        if s < _NS - 1:
            acc_b = acc_b + jnp.dot(p1[s + 1][:, 0:18], w2nb_ref[...],
                                    preferred_element_type=f32)
        a = jnp.maximum(acc_a + b2_ref[...], 0.0)
        bd = jnp.maximum(acc_b + b2_ref[...], 0.0)
        p2.append(jnp.maximum(a, bd))

    accs = [jnp.zeros((bb, _HID), f32) for _ in range(4)]
    for s in range(_NS):
        accs[(2 * s) % 4] = accs[(2 * s) % 4] + jnp.dot(
            p2[s][:, 0:36], wf1_ref[2 * s], preferred_element_type=f32)
        accs[(2 * s + 1) % 4] = accs[(2 * s + 1) % 4] + jnp.dot(
            p2[s][:, 36:72], wf1_ref[2 * s + 1], preferred_element_type=f32)
    z1 = jnp.maximum(
        (accs[0] + accs[1]) + (accs[2] + accs[3]) + bf1_ref[...], 0.0)

    out_ref[...] = (jnp.dot(z1, wf2_ref[...], preferred_element_type=f32)
                    + bf2_ref[...])


def _conv1_weights(w1):
    w1t = jnp.transpose(w1.astype(jnp.float32), (2, 1, 0))
    ma = jnp.zeros((_G, _CIN, 4, _C1), jnp.float32)
    mb = jnp.zeros((_G, _CIN, 4, _C1), jnp.float32)
    for col in range(4):
        for k in range(3):
            ti_a = 2 * col + k - 1
            if 0 <= ti_a < _G:
                ma = ma.at[ti_a, :, col, :].set(w1t[k])
            ti_b = 2 * col + 1 + k - 1
            if 0 <= ti_b < _G:
                mb = mb.at[ti_b, :, col, :].set(w1t[k])
    pa = jnp.zeros((_CIN, 4, _C1), jnp.float32)
    pa = pa.at[:, 0, :].set(w1t[0])
    nb = jnp.zeros((_CIN, 4, _C1), jnp.float32)
    nb = nb.at[:, 3, :].set(w1t[2])
    return (ma.reshape(_G * _CIN, 4 * _C1), pa.reshape(_CIN, 4 * _C1),
            mb.reshape(_G * _CIN, 4 * _C1), nb.reshape(_CIN, 4 * _C1))


def _conv2_weights(w2):
    w2t = jnp.transpose(w2.astype(jnp.float32), (2, 1, 0))
    ma = jnp.zeros((4, _C1, 2, _C2), jnp.float32)
    mb = jnp.zeros((4, _C1, 2, _C2), jnp.float32)
    for col in range(2):
        for k in range(3):
            tp_a = 2 * col + k - 1
            if 0 <= tp_a < 4:
                ma = ma.at[tp_a, :, col, :].set(w2t[k])
            tp_b = 2 * col + 1 + k - 1
            if 0 <= tp_b < 4:
                mb = mb.at[tp_b, :, col, :].set(w2t[k])
    pa = jnp.zeros((_C1, 2, _C2), jnp.float32)
    pa = pa.at[:, 0, :].set(w2t[0])
    nb = jnp.zeros((_C1, 2, _C2), jnp.float32)
    nb = nb.at[:, 1, :].set(w2t[2])
    return (ma.reshape(4 * _C1, 2 * _C2), pa.reshape(_C1, 2 * _C2),
            mb.reshape(4 * _C1, 2 * _C2), nb.reshape(_C1, 2 * _C2))


def kernel(x, w1, b1, w2, b2, wf1, bf1, wf2, bf2, block_b=1024):
    b = x.shape[0]
    assert x.shape == (b, _T, _CIN)
    assert block_b % 8 == 0
    b_pad = ((b + block_b - 1) // block_b) * block_b
    nblk = b_pad // block_b

    xf = x.astype(jnp.float32)
    if b_pad != b:
        xf = jnp.pad(xf, ((0, b_pad - b), (0, 0), (0, 0)))
    x2 = xf.reshape(b_pad, _T * _CIN).astype(jnp.bfloat16)

    w1ma, w1pa, w1mb, w1nb = _conv1_weights(w1)
    w2ma, w2pa, w2mb, w2nb = _conv2_weights(w2)
    wf1k = (wf1.astype(jnp.float32)
            .reshape(_HID, _C2, _T4).transpose(2, 1, 0))
    wf2k = wf2.astype(jnp.float32).T
    b1k = jnp.tile(b1.astype(jnp.float32), (4,))[None, :]
    b2k = jnp.tile(b2.astype(jnp.float32), (2,))[None, :]
    bf1k = bf1.astype(jnp.float32)[None, :]
    bf2k = bf2.astype(jnp.float32)[None, :]

    cm = lambda i: (0, 0)
    out = pl.pallas_call(
        _har_kernel,
        out_shape=jax.ShapeDtypeStruct((b_pad, _NCLS), jnp.float32),
        grid=(nblk,),
        in_specs=[
            pl.BlockSpec((block_b, _T * _CIN), lambda i: (i, 0)),
            pl.BlockSpec(w1ma.shape, cm),
            pl.BlockSpec(w1pa.shape, cm),
            pl.BlockSpec(w1mb.shape, cm),
            pl.BlockSpec(w1nb.shape, cm),
            pl.BlockSpec(b1k.shape, cm),
            pl.BlockSpec(w2ma.shape, cm),
            pl.BlockSpec(w2pa.shape, cm),
            pl.BlockSpec(w2mb.shape, cm),
            pl.BlockSpec(w2nb.shape, cm),
            pl.BlockSpec(b2k.shape, cm),
            pl.BlockSpec(wf1k.shape, lambda i: (0, 0, 0)),
            pl.BlockSpec(bf1k.shape, cm),
            pl.BlockSpec(wf2k.shape, cm),
            pl.BlockSpec(bf2k.shape, cm),
        ],
        out_specs=pl.BlockSpec((block_b, _NCLS), lambda i: (i, 0)),
        compiler_params=pltpu.CompilerParams(
            dimension_semantics=("parallel",),
            vmem_limit_bytes=64 * 1024 * 1024),
    )(x2, w1ma, w1pa, w1mb, w1nb, b1k, w2ma, w2pa, w2mb, w2nb, b2k,
      wf1k, bf1k, wf2k, bf2k)
    return out[:b]


# speedup vs baseline: 1.0733x; 1.0733x over previous
import jax
import jax.numpy as jnp
from jax import lax
from jax.experimental import pallas as pl
from jax.experimental.pallas import tpu as pltpu

_T = 128
_CIN = 9
_C1 = 18
_C2 = 36
_HID = 128
_G = 8
_NS = _T // _G
_T4 = _T // 4
_NCLS = 6


def _har_kernel(x_ref, w1ma_ref, w1pa_ref, w1mb_ref, w1nb_ref, b1_ref,
                w2ma_ref, w2pa_ref, w2mb_ref, w2nb_ref, b2_ref,
                wf1_ref, bf1_ref, wf2_ref, bf2_ref, out_ref):
    bb = x_ref.shape[0]
    f32 = jnp.float32

    xw = x_ref[...].astype(f32)
    xs = [xw[:, 72 * s:72 * (s + 1)] for s in range(_NS)]

    p1 = []
    for s in range(_NS):
        acc_a = jnp.dot(xs[s], w1ma_ref[...], preferred_element_type=f32)
        if s > 0:
            acc_a = acc_a + jnp.dot(xs[s - 1][:, 63:72], w1pa_ref[...],
                                    preferred_element_type=f32)
        acc_b = jnp.dot(xs[s], w1mb_ref[...], preferred_element_type=f32)
        if s < _NS - 1:
            acc_b = acc_b + jnp.dot(xs[s + 1][:, 0:9], w1nb_ref[...],
                                    preferred_element_type=f32)
        p1.append(jnp.maximum(jnp.maximum(acc_a, acc_b) + b1_ref[...], 0.0))

    p2 = []
    for s in range(_NS):
        acc_a = jnp.dot(p1[s], w2ma_ref[...], preferred_element_type=f32)
        if s > 0:
            acc_a = acc_a + jnp.dot(p1[s - 1][:, 54:72], w2pa_ref[...],
                                    preferred_element_type=f32)
        acc_b = jnp.dot(p1[s], w2mb_ref[...], preferred_element_type=f32)
        if s < _NS - 1:
            acc_b = acc_b + jnp.dot(p1[s + 1][:, 0:18], w2nb_ref[...],
                                    preferred_element_type=f32)
        p2.append(jnp.maximum(jnp.maximum(acc_a, acc_b) + b2_ref[...], 0.0))

    accs = [jnp.zeros((bb, _HID), f32) for _ in range(4)]
    for s in range(_NS):
        accs[s % 4] = accs[s % 4] + jnp.dot(
            p2[s], wf1_ref[s], preferred_element_type=f32)
    z1 = jnp.maximum(
        (accs[0] + accs[1]) + (accs[2] + accs[3]) + bf1_ref[...], 0.0)

    out_ref[...] = (jnp.dot(z1, wf2_ref[...], preferred_element_type=f32)
                    + bf2_ref[...])


def _conv1_weights(w1):
    w1t = jnp.transpose(w1.astype(jnp.float32), (2, 1, 0))
    ma = jnp.zeros((_G, _CIN, 4, _C1), jnp.float32)
    mb = jnp.zeros((_G, _CIN, 4, _C1), jnp.float32)
    for col in range(4):
        for k in range(3):
            ti_a = 2 * col + k - 1
            if 0 <= ti_a < _G:
                ma = ma.at[ti_a, :, col, :].set(w1t[k])
            ti_b = 2 * col + 1 + k - 1
            if 0 <= ti_b < _G:
                mb = mb.at[ti_b, :, col, :].set(w1t[k])
    pa = jnp.zeros((_CIN, 4, _C1), jnp.float32)
    pa = pa.at[:, 0, :].set(w1t[0])
    nb = jnp.zeros((_CIN, 4, _C1), jnp.float32)
    nb = nb.at[:, 3, :].set(w1t[2])
    return (ma.reshape(_G * _CIN, 4 * _C1), pa.reshape(_CIN, 4 * _C1),
            mb.reshape(_G * _CIN, 4 * _C1), nb.reshape(_CIN, 4 * _C1))


def _conv2_weights(w2):
    w2t = jnp.transpose(w2.astype(jnp.float32), (2, 1, 0))
    ma = jnp.zeros((4, _C1, 2, _C2), jnp.float32)
    mb = jnp.zeros((4, _C1, 2, _C2), jnp.float32)
    for col in range(2):
        for k in range(3):
            tp_a = 2 * col + k - 1
            if 0 <= tp_a < 4:
                ma = ma.at[tp_a, :, col, :].set(w2t[k])
            tp_b = 2 * col + 1 + k - 1
            if 0 <= tp_b < 4:
                mb = mb.at[tp_b, :, col, :].set(w2t[k])
    pa = jnp.zeros((_C1, 2, _C2), jnp.float32)
    pa = pa.at[:, 0, :].set(w2t[0])
    nb = jnp.zeros((_C1, 2, _C2), jnp.float32)
    nb = nb.at[:, 1, :].set(w2t[2])
    return (ma.reshape(4 * _C1, 2 * _C2), pa.reshape(_C1, 2 * _C2),
            mb.reshape(4 * _C1, 2 * _C2), nb.reshape(_C1, 2 * _C2))


def kernel(x, w1, b1, w2, b2, wf1, bf1, wf2, bf2, block_b=2048):
    b = x.shape[0]
    assert x.shape == (b, _T, _CIN)
    assert block_b % 8 == 0
    b_pad = ((b + block_b - 1) // block_b) * block_b
    nblk = b_pad // block_b

    xf = x.astype(jnp.float32)
    if b_pad != b:
        xf = jnp.pad(xf, ((0, b_pad - b), (0, 0), (0, 0)))
    x2 = xf.reshape(b_pad, _T * _CIN).astype(jnp.bfloat16)

    w1ma, w1pa, w1mb, w1nb = _conv1_weights(w1)
    w2ma, w2pa, w2mb, w2nb = _conv2_weights(w2)
    wf1k = (wf1.astype(jnp.float32)
            .reshape(_HID, _C2, _T4).transpose(2, 1, 0)
            .reshape(_NS, 2 * _C2, _HID))
    wf2k = wf2.astype(jnp.float32).T
    b1k = jnp.tile(b1.astype(jnp.float32), (4,))[None, :]
    b2k = jnp.tile(b2.astype(jnp.float32), (2,))[None, :]
    bf1k = bf1.astype(jnp.float32)[None, :]
    bf2k = bf2.astype(jnp.float32)[None, :]

    cm = lambda i: (0, 0)
    out = pl.pallas_call(
        _har_kernel,
        out_shape=jax.ShapeDtypeStruct((b_pad, _NCLS), jnp.float32),
        grid=(nblk,),
        in_specs=[
            pl.BlockSpec((block_b, _T * _CIN), lambda i: (i, 0)),
            pl.BlockSpec(w1ma.shape, cm),
            pl.BlockSpec(w1pa.shape, cm),
            pl.BlockSpec(w1mb.shape, cm),
            pl.BlockSpec(w1nb.shape, cm),
            pl.BlockSpec(b1k.shape, cm),
            pl.BlockSpec(w2ma.shape, cm),
            pl.BlockSpec(w2pa.shape, cm),
            pl.BlockSpec(w2mb.shape, cm),
            pl.BlockSpec(w2nb.shape, cm),
            pl.BlockSpec(b2k.shape, cm),
            pl.BlockSpec(wf1k.shape, lambda i: (0, 0, 0)),
            pl.BlockSpec(bf1k.shape, cm),
            pl.BlockSpec(wf2k.shape, cm),
            pl.BlockSpec(bf2k.shape, cm),
        ],
        out_specs=pl.BlockSpec((block_b, _NCLS), lambda i: (i, 0)),
        compiler_params=pltpu.CompilerParams(
            dimension_semantics=("parallel",),
            vmem_limit_bytes=64 * 1024 * 1024),
    )(x2, w1ma, w1pa, w1mb, w1nb, b1k, w2ma, w2pa, w2mb, w2nb, b2k,
      wf1k, bf1k, wf2k, bf2k)
    return out[:b]


# speedup vs baseline: 1.0837x; 1.0097x over previous
import jax
import jax.numpy as jnp
from jax import lax
from jax.experimental import pallas as pl
from jax.experimental.pallas import tpu as pltpu

_T = 128
_CIN = 9
_C1 = 18
_C2 = 36
_HID = 128
_G = 8
_NS = _T // _G
_T4 = _T // 4
_NCLS = 6


def _har_kernel(x_ref, w1ma_ref, w1pa_ref, w1mb_ref, w1nb_ref, b1_ref,
                w2ma_ref, w2pa_ref, w2mb_ref, w2nb_ref, b2_ref,
                wf1_ref, bf1_ref, wf2_ref, bf2_ref, out_ref):
    bb = x_ref.shape[0]
    f32 = jnp.float32

    xw = x_ref[...]
    xs = [xw[:, 72 * s:72 * (s + 1)] for s in range(_NS)]

    p1 = []
    for s in range(_NS):
        acc_a = jnp.dot(xs[s], w1ma_ref[...], preferred_element_type=f32)
        if s > 0:
            acc_a = acc_a + jnp.dot(xs[s - 1][:, 63:72], w1pa_ref[...],
                                    preferred_element_type=f32)
        acc_b = jnp.dot(xs[s], w1mb_ref[...], preferred_element_type=f32)
        if s < _NS - 1:
            acc_b = acc_b + jnp.dot(xs[s + 1][:, 0:9], w1nb_ref[...],
                                    preferred_element_type=f32)
        p1.append(jnp.maximum(jnp.maximum(acc_a, acc_b) + b1_ref[...], 0.0))

    p2 = []
    for s in range(_NS):
        acc_a = jnp.dot(p1[s], w2ma_ref[...], preferred_element_type=f32)
        if s > 0:
            acc_a = acc_a + jnp.dot(p1[s - 1][:, 54:72], w2pa_ref[...],
                                    preferred_element_type=f32)
        acc_b = jnp.dot(p1[s], w2mb_ref[...], preferred_element_type=f32)
        if s < _NS - 1:
            acc_b = acc_b + jnp.dot(p1[s + 1][:, 0:18], w2nb_ref[...],
                                    preferred_element_type=f32)
        p2.append(jnp.maximum(jnp.maximum(acc_a, acc_b) + b2_ref[...], 0.0))

    accs = [jnp.zeros((bb, _HID), f32) for _ in range(4)]
    for s in range(_NS):
        accs[s % 4] = accs[s % 4] + jnp.dot(
            p2[s], wf1_ref[s], preferred_element_type=f32)
    z1 = jnp.maximum(
        (accs[0] + accs[1]) + (accs[2] + accs[3]) + bf1_ref[...], 0.0)

    out_ref[...] = (jnp.dot(z1, wf2_ref[...], preferred_element_type=f32)
                    + bf2_ref[...])


def _conv1_weights(w1):
    w1t = jnp.transpose(w1.astype(jnp.float32), (2, 1, 0))
    ma = jnp.zeros((_G, _CIN, 4, _C1), jnp.float32)
    mb = jnp.zeros((_G, _CIN, 4, _C1), jnp.float32)
    for col in range(4):
        for k in range(3):
            ti_a = 2 * col + k - 1
            if 0 <= ti_a < _G:
                ma = ma.at[ti_a, :, col, :].set(w1t[k])
            ti_b = 2 * col + 1 + k - 1
            if 0 <= ti_b < _G:
                mb = mb.at[ti_b, :, col, :].set(w1t[k])
    pa = jnp.zeros((_CIN, 4, _C1), jnp.float32)
    pa = pa.at[:, 0, :].set(w1t[0])
    nb = jnp.zeros((_CIN, 4, _C1), jnp.float32)
    nb = nb.at[:, 3, :].set(w1t[2])
    return (ma.reshape(_G * _CIN, 4 * _C1), pa.reshape(_CIN, 4 * _C1),
            mb.reshape(_G * _CIN, 4 * _C1), nb.reshape(_CIN, 4 * _C1))


def _conv2_weights(w2):
    w2t = jnp.transpose(w2.astype(jnp.float32), (2, 1, 0))
    ma = jnp.zeros((4, _C1, 2, _C2), jnp.float32)
    mb = jnp.zeros((4, _C1, 2, _C2), jnp.float32)
    for col in range(2):
        for k in range(3):
            tp_a = 2 * col + k - 1
            if 0 <= tp_a < 4:
                ma = ma.at[tp_a, :, col, :].set(w2t[k])
            tp_b = 2 * col + 1 + k - 1
            if 0 <= tp_b < 4:
                mb = mb.at[tp_b, :, col, :].set(w2t[k])
    pa = jnp.zeros((_C1, 2, _C2), jnp.float32)
    pa = pa.at[:, 0, :].set(w2t[0])
    nb = jnp.zeros((_C1, 2, _C2), jnp.float32)
    nb = nb.at[:, 1, :].set(w2t[2])
    return (ma.reshape(4 * _C1, 2 * _C2), pa.reshape(_C1, 2 * _C2),
            mb.reshape(4 * _C1, 2 * _C2), nb.reshape(_C1, 2 * _C2))


def kernel(x, w1, b1, w2, b2, wf1, bf1, wf2, bf2, block_b=2048):
    b = x.shape[0]
    assert x.shape == (b, _T, _CIN)
    assert block_b % 8 == 0
    b_pad = ((b + block_b - 1) // block_b) * block_b
    nblk = b_pad // block_b

    xf = x.astype(jnp.float32)
    if b_pad != b:
        xf = jnp.pad(xf, ((0, b_pad - b), (0, 0), (0, 0)))
    x2 = xf.reshape(b_pad, _T * _CIN).astype(jnp.bfloat16)

    w1ma, w1pa, w1mb, w1nb = [w.astype(jnp.bfloat16)
                               for w in _conv1_weights(w1)]
    w2ma, w2pa, w2mb, w2nb = _conv2_weights(w2)
    wf1k = (wf1.astype(jnp.float32)
            .reshape(_HID, _C2, _T4).transpose(2, 1, 0)
            .reshape(_NS, 2 * _C2, _HID))
    wf2k = wf2.astype(jnp.float32).T
    b1k = jnp.tile(b1.astype(jnp.float32), (4,))[None, :]
    b2k = jnp.tile(b2.astype(jnp.float32), (2,))[None, :]
    bf1k = bf1.astype(jnp.float32)[None, :]
    bf2k = bf2.astype(jnp.float32)[None, :]

    cm = lambda i: (0, 0)
    out = pl.pallas_call(
        _har_kernel,
        out_shape=jax.ShapeDtypeStruct((b_pad, _NCLS), jnp.float32),
        grid=(nblk,),
        in_specs=[
            pl.BlockSpec((block_b, _T * _CIN), lambda i: (i, 0)),
            pl.BlockSpec(w1ma.shape, cm),
            pl.BlockSpec(w1pa.shape, cm),
            pl.BlockSpec(w1mb.shape, cm),
            pl.BlockSpec(w1nb.shape, cm),
            pl.BlockSpec(b1k.shape, cm),
            pl.BlockSpec(w2ma.shape, cm),
            pl.BlockSpec(w2pa.shape, cm),
            pl.BlockSpec(w2mb.shape, cm),
            pl.BlockSpec(w2nb.shape, cm),
            pl.BlockSpec(b2k.shape, cm),
            pl.BlockSpec(wf1k.shape, lambda i: (0, 0, 0)),
            pl.BlockSpec(bf1k.shape, cm),
            pl.BlockSpec(wf2k.shape, cm),
            pl.BlockSpec(bf2k.shape, cm),
        ],
        out_specs=pl.BlockSpec((block_b, _NCLS), lambda i: (i, 0)),
        compiler_params=pltpu.CompilerParams(
            dimension_semantics=("parallel",),
            vmem_limit_bytes=64 * 1024 * 1024),
    )(x2, w1ma, w1pa, w1mb, w1nb, b1k, w2ma, w2pa, w2mb, w2nb, b2k,
      wf1k, bf1k, wf2k, bf2k)
    return out[:b]


# speedup vs baseline: 1.0839x; 1.0001x over previous
import jax
import jax.numpy as jnp
from jax import lax
from jax.experimental import pallas as pl
from jax.experimental.pallas import tpu as pltpu

_T = 128
_CIN = 9
_C1 = 18
_C2 = 36
_HID = 128
_G = 8
_NS = _T // _G
_T4 = _T // 4
_NCLS = 6


def _har_kernel(x_ref, w1ma_ref, w1pa_ref, w1mb_ref, w1nb_ref, b1_ref,
                w2ma_ref, w2pa_ref, w2mb_ref, w2nb_ref, b2_ref,
                wf1_ref, bf1_ref, wf2_ref, bf2_ref, out_ref):
    bb = x_ref.shape[0]
    f32 = jnp.float32

    xw = x_ref[...]
    xs = [xw[:, 72 * s:72 * (s + 1)] for s in range(_NS)]

    p1 = []
    for s in range(_NS):
        acc_a = jnp.dot(xs[s], w1ma_ref[...], preferred_element_type=f32)
        if s > 0:
            acc_a = acc_a + jnp.dot(xs[s - 1][:, 63:72], w1pa_ref[...],
                                    preferred_element_type=f32)
        acc_b = jnp.dot(xs[s], w1mb_ref[...], preferred_element_type=f32)
        if s < _NS - 1:
            acc_b = acc_b + jnp.dot(xs[s + 1][:, 0:9], w1nb_ref[...],
                                    preferred_element_type=f32)
        p1.append(jnp.maximum(jnp.maximum(acc_a, acc_b) + b1_ref[...],
                              0.0).astype(jnp.bfloat16))

    p2 = []
    for s in range(_NS):
        acc_a = jnp.dot(p1[s], w2ma_ref[...], preferred_element_type=f32)
        if s > 0:
            acc_a = acc_a + jnp.dot(p1[s - 1][:, 54:72], w2pa_ref[...],
                                    preferred_element_type=f32)
        acc_b = jnp.dot(p1[s], w2mb_ref[...], preferred_element_type=f32)
        if s < _NS - 1:
            acc_b = acc_b + jnp.dot(p1[s + 1][:, 0:18], w2nb_ref[...],
                                    preferred_element_type=f32)
        p2.append(jnp.maximum(jnp.maximum(acc_a, acc_b) + b2_ref[...],
                              0.0).astype(jnp.bfloat16))

    accs = [jnp.zeros((bb, _HID), f32) for _ in range(4)]
    for s in range(_NS):
        accs[s % 4] = accs[s % 4] + jnp.dot(
            p2[s], wf1_ref[s], preferred_element_type=f32)
    z1 = jnp.maximum(
        (accs[0] + accs[1]) + (accs[2] + accs[3]) + bf1_ref[...], 0.0)

    out_ref[...] = (jnp.dot(z1, wf2_ref[...], preferred_element_type=f32)
                    + bf2_ref[...])


def _conv1_weights(w1):
    w1t = jnp.transpose(w1.astype(jnp.float32), (2, 1, 0))
    ma = jnp.zeros((_G, _CIN, 4, _C1), jnp.float32)
    mb = jnp.zeros((_G, _CIN, 4, _C1), jnp.float32)
    for col in range(4):
        for k in range(3):
            ti_a = 2 * col + k - 1
            if 0 <= ti_a < _G:
                ma = ma.at[ti_a, :, col, :].set(w1t[k])
            ti_b = 2 * col + 1 + k - 1
            if 0 <= ti_b < _G:
                mb = mb.at[ti_b, :, col, :].set(w1t[k])
    pa = jnp.zeros((_CIN, 4, _C1), jnp.float32)
    pa = pa.at[:, 0, :].set(w1t[0])
    nb = jnp.zeros((_CIN, 4, _C1), jnp.float32)
    nb = nb.at[:, 3, :].set(w1t[2])
    return (ma.reshape(_G * _CIN, 4 * _C1), pa.reshape(_CIN, 4 * _C1),
            mb.reshape(_G * _CIN, 4 * _C1), nb.reshape(_CIN, 4 * _C1))


def _conv2_weights(w2):
    w2t = jnp.transpose(w2.astype(jnp.float32), (2, 1, 0))
    ma = jnp.zeros((4, _C1, 2, _C2), jnp.float32)
    mb = jnp.zeros((4, _C1, 2, _C2), jnp.float32)
    for col in range(2):
        for k in range(3):
            tp_a = 2 * col + k - 1
            if 0 <= tp_a < 4:
                ma = ma.at[tp_a, :, col, :].set(w2t[k])
            tp_b = 2 * col + 1 + k - 1
            if 0 <= tp_b < 4:
                mb = mb.at[tp_b, :, col, :].set(w2t[k])
    pa = jnp.zeros((_C1, 2, _C2), jnp.float32)
    pa = pa.at[:, 0, :].set(w2t[0])
    nb = jnp.zeros((_C1, 2, _C2), jnp.float32)
    nb = nb.at[:, 1, :].set(w2t[2])
    return (ma.reshape(4 * _C1, 2 * _C2), pa.reshape(_C1, 2 * _C2),
            mb.reshape(4 * _C1, 2 * _C2), nb.reshape(_C1, 2 * _C2))


def kernel(x, w1, b1, w2, b2, wf1, bf1, wf2, bf2, block_b=2048):
    b = x.shape[0]
    assert x.shape == (b, _T, _CIN)
    assert block_b % 8 == 0
    b_pad = ((b + block_b - 1) // block_b) * block_b
    nblk = b_pad // block_b

    xf = x.astype(jnp.float32)
    if b_pad != b:
        xf = jnp.pad(xf, ((0, b_pad - b), (0, 0), (0, 0)))
    x2 = xf.reshape(b_pad, _T * _CIN).astype(jnp.bfloat16)

    w1ma, w1pa, w1mb, w1nb = [w.astype(jnp.bfloat16)
                               for w in _conv1_weights(w1)]
    w2ma, w2pa, w2mb, w2nb = [w.astype(jnp.bfloat16)
                               for w in _conv2_weights(w2)]
    wf1k = (wf1.astype(jnp.float32)
            .reshape(_HID, _C2, _T4).transpose(2, 1, 0)
            .reshape(_NS, 2 * _C2, _HID).astype(jnp.bfloat16))
    wf2k = wf2.astype(jnp.float32).T
    b1k = jnp.tile(b1.astype(jnp.float32), (4,))[None, :]
    b2k = jnp.tile(b2.astype(jnp.float32), (2,))[None, :]
    bf1k = bf1.astype(jnp.float32)[None, :]
    bf2k = bf2.astype(jnp.float32)[None, :]

    cm = lambda i: (0, 0)
    out = pl.pallas_call(
        _har_kernel,
        out_shape=jax.ShapeDtypeStruct((b_pad, _NCLS), jnp.float32),
        grid=(nblk,),
        in_specs=[
            pl.BlockSpec((block_b, _T * _CIN), lambda i: (i, 0)),
            pl.BlockSpec(w1ma.shape, cm),
            pl.BlockSpec(w1pa.shape, cm),
            pl.BlockSpec(w1mb.shape, cm),
            pl.BlockSpec(w1nb.shape, cm),
            pl.BlockSpec(b1k.shape, cm),
            pl.BlockSpec(w2ma.shape, cm),
            pl.BlockSpec(w2pa.shape, cm),
            pl.BlockSpec(w2mb.shape, cm),
            pl.BlockSpec(w2nb.shape, cm),
            pl.BlockSpec(b2k.shape, cm),
            pl.BlockSpec(wf1k.shape, lambda i: (0, 0, 0)),
            pl.BlockSpec(bf1k.shape, cm),
            pl.BlockSpec(wf2k.shape, cm),
            pl.BlockSpec(bf2k.shape, cm),
        ],
        out_specs=pl.BlockSpec((block_b, _NCLS), lambda i: (i, 0)),
        compiler_params=pltpu.CompilerParams(
            dimension_semantics=("parallel",),
            vmem_limit_bytes=64 * 1024 * 1024),
    )(x2, w1ma, w1pa, w1mb, w1nb, b1k, w2ma, w2pa, w2mb, w2nb, b2k,
      wf1k, bf1k, wf2k, bf2k)
    return out[:b]


# speedup vs baseline: 1.0844x; 1.0005x over previous
import jax
import jax.numpy as jnp
from jax import lax
from jax.experimental import pallas as pl
from jax.experimental.pallas import tpu as pltpu

_T = 128
_CIN = 9
_C1 = 18
_C2 = 36
_HID = 128
_G = 8
_NS = _T // _G
_T4 = _T // 4
_NCLS = 6


def _har_kernel(x_ref, w1ma_ref, w1pa_ref, w1mb_ref, w1nb_ref, b1_ref,
                w2ma_ref, w2pa_ref, w2mb_ref, w2nb_ref, b2_ref,
                wf1_ref, bf1_ref, wf2_ref, bf2_ref, out_ref):
    bb = x_ref.shape[0]
    f32 = jnp.float32

    xw = x_ref[...]
    xs = [xw[:, 72 * s:72 * (s + 1)] for s in range(_NS)]

    p1 = []
    for s in range(_NS):
        acc_a = jnp.dot(xs[s], w1ma_ref[...], preferred_element_type=f32)
        if s > 0:
            acc_a = acc_a + jnp.dot(xs[s - 1][:, 63:72], w1pa_ref[...],
                                    preferred_element_type=f32)
        acc_b = jnp.dot(xs[s], w1mb_ref[...], preferred_element_type=f32)
        if s < _NS - 1:
            acc_b = acc_b + jnp.dot(xs[s + 1][:, 0:9], w1nb_ref[...],
                                    preferred_element_type=f32)
        p1.append(jnp.maximum(jnp.maximum(acc_a, acc_b) + b1_ref[...], 0.0))

    p2 = []
    for s in range(_NS):
        acc_a = jnp.dot(p1[s], w2ma_ref[...], preferred_element_type=f32)
        if s > 0:
            acc_a = acc_a + jnp.dot(p1[s - 1][:, 54:72], w2pa_ref[...],
                                    preferred_element_type=f32)
        acc_b = jnp.dot(p1[s], w2mb_ref[...], preferred_element_type=f32)
        if s < _NS - 1:
            acc_b = acc_b + jnp.dot(p1[s + 1][:, 0:18], w2nb_ref[...],
                                    preferred_element_type=f32)
        p2.append(jnp.maximum(jnp.maximum(acc_a, acc_b) + b2_ref[...], 0.0))

    accs = [jnp.zeros((bb, _HID), f32) for _ in range(4)]
    for s in range(_NS):
        accs[s % 4] = accs[s % 4] + jnp.dot(
            p2[s], wf1_ref[s], preferred_element_type=f32)
    z1 = jnp.maximum(
        (accs[0] + accs[1]) + (accs[2] + accs[3]) + bf1_ref[...], 0.0)

    out_ref[...] = (jnp.dot(z1, wf2_ref[...], preferred_element_type=f32)
                    + bf2_ref[...])


def _conv1_weights(w1):
    w1t = jnp.transpose(w1.astype(jnp.float32), (2, 1, 0))
    ma = jnp.zeros((_G, _CIN, 4, _C1), jnp.float32)
    mb = jnp.zeros((_G, _CIN, 4, _C1), jnp.float32)
    for col in range(4):
        for k in range(3):
            ti_a = 2 * col + k - 1
            if 0 <= ti_a < _G:
                ma = ma.at[ti_a, :, col, :].set(w1t[k])
            ti_b = 2 * col + 1 + k - 1
            if 0 <= ti_b < _G:
                mb = mb.at[ti_b, :, col, :].set(w1t[k])
    pa = jnp.zeros((_CIN, 4, _C1), jnp.float32)
    pa = pa.at[:, 0, :].set(w1t[0])
    nb = jnp.zeros((_CIN, 4, _C1), jnp.float32)
    nb = nb.at[:, 3, :].set(w1t[2])
    return (ma.reshape(_G * _CIN, 4 * _C1), pa.reshape(_CIN, 4 * _C1),
            mb.reshape(_G * _CIN, 4 * _C1), nb.reshape(_CIN, 4 * _C1))


def _conv2_weights(w2):
    w2t = jnp.transpose(w2.astype(jnp.float32), (2, 1, 0))
    ma = jnp.zeros((4, _C1, 2, _C2), jnp.float32)
    mb = jnp.zeros((4, _C1, 2, _C2), jnp.float32)
    for col in range(2):
        for k in range(3):
            tp_a = 2 * col + k - 1
            if 0 <= tp_a < 4:
                ma = ma.at[tp_a, :, col, :].set(w2t[k])
            tp_b = 2 * col + 1 + k - 1
            if 0 <= tp_b < 4:
                mb = mb.at[tp_b, :, col, :].set(w2t[k])
    pa = jnp.zeros((_C1, 2, _C2), jnp.float32)
    pa = pa.at[:, 0, :].set(w2t[0])
    nb = jnp.zeros((_C1, 2, _C2), jnp.float32)
    nb = nb.at[:, 1, :].set(w2t[2])
    return (ma.reshape(4 * _C1, 2 * _C2), pa.reshape(_C1, 2 * _C2),
            mb.reshape(4 * _C1, 2 * _C2), nb.reshape(_C1, 2 * _C2))


def kernel(x, w1, b1, w2, b2, wf1, bf1, wf2, bf2, block_b=2048):
    b = x.shape[0]
    assert x.shape == (b, _T, _CIN)
    assert block_b % 8 == 0
    b_pad = ((b + block_b - 1) // block_b) * block_b
    nblk = b_pad // block_b

    xf = x.astype(jnp.float32)
    if b_pad != b:
        xf = jnp.pad(xf, ((0, b_pad - b), (0, 0), (0, 0)))
    x2 = xf.reshape(b_pad, _T * _CIN).astype(jnp.bfloat16)

    w1ma, w1pa, w1mb, w1nb = [w.astype(jnp.bfloat16)
                               for w in _conv1_weights(w1)]
    w2ma, w2pa, w2mb, w2nb = _conv2_weights(w2)
    wf1k = (wf1.astype(jnp.float32)
            .reshape(_HID, _C2, _T4).transpose(2, 1, 0)
            .reshape(_NS, 2 * _C2, _HID))
    wf2k = wf2.astype(jnp.float32).T
    b1k = jnp.tile(b1.astype(jnp.float32), (4,))[None, :]
    b2k = jnp.tile(b2.astype(jnp.float32), (2,))[None, :]
    bf1k = bf1.astype(jnp.float32)[None, :]
    bf2k = bf2.astype(jnp.float32)[None, :]

    cm = lambda i: (0, 0)
    out = pl.pallas_call(
        _har_kernel,
        out_shape=jax.ShapeDtypeStruct((b_pad, _NCLS), jnp.float32),
        grid=(nblk,),
        in_specs=[
            pl.BlockSpec((block_b, _T * _CIN), lambda i: (i, 0)),
            pl.BlockSpec(w1ma.shape, cm),
            pl.BlockSpec(w1pa.shape, cm),
            pl.BlockSpec(w1mb.shape, cm),
            pl.BlockSpec(w1nb.shape, cm),
            pl.BlockSpec(b1k.shape, cm),
            pl.BlockSpec(w2ma.shape, cm),
            pl.BlockSpec(w2pa.shape, cm),
            pl.BlockSpec(w2mb.shape, cm),
            pl.BlockSpec(w2nb.shape, cm),
            pl.BlockSpec(b2k.shape, cm),
            pl.BlockSpec(wf1k.shape, lambda i: (0, 0, 0)),
            pl.BlockSpec(bf1k.shape, cm),
            pl.BlockSpec(wf2k.shape, cm),
            pl.BlockSpec(bf2k.shape, cm),
        ],
        out_specs=pl.BlockSpec((block_b, _NCLS), lambda i: (i, 0)),
        compiler_params=pltpu.CompilerParams(
            dimension_semantics=("parallel",),
            vmem_limit_bytes=64 * 1024 * 1024),
    )(x2, w1ma, w1pa, w1mb, w1nb, b1k, w2ma, w2pa, w2mb, w2nb, b2k,
      wf1k, bf1k, wf2k, bf2k)
    return out[:b]
